# baseline (speedup 1.0000x reference)
import jax
import jax.numpy as jnp
from jax import lax
from jax.experimental import pallas as pl
from jax.experimental.pallas import tpu as pltpu
from jax.experimental.pallas import tpu_sc as plsc

_R, _N = 128, 32768
_SC_ROWS = 64
_TC_ROWS = _R - _SC_ROWS
_L = 16
_NCH = _N // _L
_NW = 32
_NB = 256
_LEVELS = 2
_TC_BR = 16
_TC_ITERS = 12


def _sc_body(in_hbm, out_hbm, bufs, hcnt, hsum, sem_in, sem_out):
    rpw = _SC_ROWS // _NW
    cid = lax.axis_index("c")
    sid = lax.axis_index("s")
    wid = sid * 2 + cid
    lanes = lax.iota(jnp.int32, _L)
    zeros = jnp.zeros((_L,), jnp.float32)
    ones = jnp.ones((_L,), jnp.float32)

    copies_in = []
    for r in range(rpw):
        copies_in.append(
            pltpu.async_copy(in_hbm.at[wid * rpw + r], bufs.at[pl.ds(r * _N, _N)], sem_in))
    copy_out = None
    for r in range(rpw):
        row = wid * rpw + r
        buf = bufs.at[pl.ds(r * _N, _N)]
        copies_in[r].wait()

        @plsc.parallel_loop(0, _NCH, carry=jnp.full((_L,), -jnp.inf, jnp.float32), unroll=8)
        def _mx(i, acc):
            return jnp.maximum(acc, buf[pl.ds(i * _L, _L)])

        m = jnp.max(_mx)
        lo = m - 1.0
        hi = m
        cc_star = jnp.float32(1.0)
        ss_star = jnp.float32(0.0)

        for lev in range(_LEVELS):
            inv_bw = jnp.float32(2.0 ** (8 + 6 * lev))
            bw = jnp.float32(2.0 ** -(8 + 6 * lev))

            @plsc.parallel_loop(0, _NB, unroll=4)
            def _zero(b):
                hcnt[pl.ds(b * _L, _L)] = zeros
                hsum[pl.ds(b * _L, _L)] = zeros

            if lev == 0:
                @plsc.parallel_loop(0, _NCH, unroll=8)
                def _hist0(i):
                    v = buf[pl.ds(i * _L, _L)]
                    b = jnp.minimum((hi - v) * inv_bw, _NB - 1.0).astype(jnp.int32)
                    addr = b * _L + lanes
                    plsc.addupdate_scatter(hcnt, [addr], ones)
                    plsc.addupdate_scatter(hsum, [addr], v)

                c_top = jnp.float32(0.0)
                s_top = jnp.float32(0.0)
            else:
                @plsc.parallel_loop(0, _NCH, carry=(zeros, zeros), unroll=8)
                def _hist(i, carry):
                    ca, sa = carry
                    v = buf[pl.ds(i * _L, _L)]
                    b = jnp.clip((hi - v) * inv_bw, 0.0, _NB - 1.0).astype(jnp.int32)
                    addr = b * _L + lanes
                    mask = (v <= hi) & (v >= lo)
                    plsc.addupdate_scatter(hcnt, [addr], ones, mask=mask)
                    plsc.addupdate_scatter(hsum, [addr], v, mask=mask)
                    above = v > hi
                    return (ca + jnp.where(above, 1.0, 0.0),
                            sa + jnp.where(above, v, 0.0))

                ca, sa = _hist
                c_top = jnp.sum(ca)
                s_top = jnp.sum(sa)

            @plsc.parallel_loop(
                0, _NB, unroll=4,
                carry=(jnp.zeros((), jnp.float32), jnp.zeros((), jnp.float32),
                       jnp.full((), _NB - 1, jnp.int32),
                       jnp.zeros((), jnp.bool_),
                       jnp.ones((), jnp.float32), jnp.zeros((), jnp.float32)))
            def _scan(b, carry):
                cc, ss, bstar, found, ccs, sss = carry
                cc = cc + jnp.sum(hcnt[pl.ds(b * _L, _L)])
                ss = ss + jnp.sum(hsum[pl.ds(b * _L, _L)])
                t_edge = hi - (b + 1).astype(jnp.float32) * bw
                f = (s_top + ss) - (c_top + cc) * t_edge - 1.0
                hit = (f >= 0.0) & jnp.logical_not(found)
                bstar = jnp.where(hit, b, bstar)
                ccs = jnp.where(hit, c_top + cc, ccs)
                sss = jnp.where(hit, s_top + ss, sss)
                return cc, ss, bstar, found | hit, ccs, sss

            cc, ss, bstar, found, cc_star, ss_star = _scan
            cc_star = jnp.where(found, cc_star, c_top + cc)
            ss_star = jnp.where(found, ss_star, s_top + ss)
            bsf = bstar.astype(jnp.float32)
            lo = hi - (bsf + 3.0) * bw
            hi = hi - (bsf - 1.0) * bw

        tau = jnp.full((_L,), ss_star - 1.0) / jnp.full((_L,), cc_star)

        @plsc.parallel_loop(0, _NCH, unroll=8)
        def _out(i):
            v = buf[pl.ds(i * _L, _L)]
            buf[pl.ds(i * _L, _L)] = jnp.maximum(v - tau, 0.0)

        if copy_out is not None:
            copy_out.wait()
        copy_out = pltpu.async_copy(buf, out_hbm.at[row], sem_out)
    copy_out.wait()


def _make_sc_kernel():
    mesh = plsc.VectorSubcoreMesh(
        core_axis_name="c", subcore_axis_name="s",
        num_cores=2, num_subcores=16)
    return pl.kernel(
        _sc_body,
        out_type=jax.ShapeDtypeStruct((_R, _N), jnp.float32),
        mesh=mesh,
        scratch_types=[
            pltpu.VMEM(((_SC_ROWS // _NW) * _N,), jnp.float32),
            pltpu.VMEM((_NB * _L,), jnp.float32),
            pltpu.VMEM((_NB * _L,), jnp.float32),
            pltpu.SemaphoreType.DMA,
            pltpu.SemaphoreType.DMA,
        ],
        compiler_params=pltpu.CompilerParams(needs_layout_passes=False),
    )


def _tc_body(x_ref, o_ref):
    x = x_ref[...]
    m = jnp.max(x, axis=-1, keepdims=True)
    lo = m - 1.0
    hi = m

    def it(_, carry):
        lo, hi = carry
        mid = 0.5 * (lo + hi)
        s = jnp.sum(jnp.maximum(x - mid, 0.0), axis=-1, keepdims=True)
        pred = s > 1.0
        return jnp.where(pred, mid, lo), jnp.where(pred, hi, mid)

    lo, hi = jax.lax.fori_loop(0, _TC_ITERS, it, (lo, hi))
    sup = x > lo
    k = jnp.sum(sup.astype(jnp.float32), axis=-1, keepdims=True)
    s = jnp.sum(jnp.where(sup, x, 0.0), axis=-1, keepdims=True)
    tau = (s - 1.0) / k
    o_ref[...] = jnp.maximum(x - tau, 0.0)


def _tc_kernel(x_full):
    off = _SC_ROWS // _TC_BR
    return pl.pallas_call(
        _tc_body,
        grid=(_TC_ROWS // _TC_BR,),
        in_specs=[pl.BlockSpec((_TC_BR, _N), lambda i: (i + off, 0))],
        out_specs=pl.BlockSpec((_TC_BR, _N), lambda i: (i, 0)),
        out_shape=jax.ShapeDtypeStruct((_TC_ROWS, _N), jnp.float32),
    )(x_full)


@jax.jit
def kernel(input):
    sc_out = _make_sc_kernel()(input)
    tc_out = _tc_kernel(input)
    return lax.dynamic_update_slice(sc_out, tc_out, (_SC_ROWS, 0))


# speedup vs baseline: 1.0043x; 1.0043x over previous
import jax
import jax.numpy as jnp
from jax import lax
from jax.experimental import pallas as pl
from jax.experimental.pallas import tpu as pltpu
from jax.experimental.pallas import tpu_sc as plsc

_R, _N = 128, 32768
_SC_ROWS = 64
_TC_ROWS = _R - _SC_ROWS
_L = 16
_NCH = _N // _L
_NW = 32
_NB = 256
_LEVELS = 2
_TC_BR = 8
_TC_ITERS = 12


def _sc_body(in_hbm, out_hbm, bufs, hcnt, hsum, sem_in, sem_out):
    rpw = _SC_ROWS // _NW
    cid = lax.axis_index("c")
    sid = lax.axis_index("s")
    wid = sid * 2 + cid
    lanes = lax.iota(jnp.int32, _L)
    zeros = jnp.zeros((_L,), jnp.float32)
    ones = jnp.ones((_L,), jnp.float32)

    copies_in = []
    for r in range(rpw):
        copies_in.append(
            pltpu.async_copy(in_hbm.at[wid * rpw + r], bufs.at[pl.ds(r * _N, _N)], sem_in))
    copy_out = None
    for r in range(rpw):
        row = wid * rpw + r
        buf = bufs.at[pl.ds(r * _N, _N)]
        copies_in[r].wait()

        @plsc.parallel_loop(0, _NCH, carry=jnp.full((_L,), -jnp.inf, jnp.float32), unroll=8)
        def _mx(i, acc):
            return jnp.maximum(acc, buf[pl.ds(i * _L, _L)])

        m = jnp.max(_mx)
        lo = m - 1.0
        hi = m
        cc_star = jnp.float32(1.0)
        ss_star = jnp.float32(0.0)

        for lev in range(_LEVELS):
            inv_bw = jnp.float32(2.0 ** (8 + 6 * lev))
            bw = jnp.float32(2.0 ** -(8 + 6 * lev))

            @plsc.parallel_loop(0, _NB, unroll=4)
            def _zero(b):
                hcnt[pl.ds(b * _L, _L)] = zeros
                hsum[pl.ds(b * _L, _L)] = zeros

            if lev == 0:
                @plsc.parallel_loop(0, _NCH, unroll=8)
                def _hist0(i):
                    v = buf[pl.ds(i * _L, _L)]
                    b = jnp.minimum((hi - v) * inv_bw, _NB - 1.0).astype(jnp.int32)
                    addr = b * _L + lanes
                    plsc.addupdate_scatter(hcnt, [addr], ones)
                    plsc.addupdate_scatter(hsum, [addr], v)

                c_top = jnp.float32(0.0)
                s_top = jnp.float32(0.0)
            else:
                @plsc.parallel_loop(0, _NCH, carry=(zeros, zeros), unroll=4)
                def _hist(i, carry):
                    ca, sa = carry
                    v = buf[pl.ds(i * _L, _L)]
                    b = jnp.clip((hi - v) * inv_bw, 0.0, _NB - 1.0).astype(jnp.int32)
                    addr = b * _L + lanes
                    mask = (v <= hi) & (v >= lo)
                    plsc.addupdate_scatter(hcnt, [addr], ones, mask=mask)
                    plsc.addupdate_scatter(hsum, [addr], v, mask=mask)
                    above = v > hi
                    return (ca + jnp.where(above, 1.0, 0.0),
                            sa + jnp.where(above, v, 0.0))

                ca, sa = _hist
                c_top = jnp.sum(ca)
                s_top = jnp.sum(sa)

            @plsc.parallel_loop(
                0, _NB, unroll=4,
                carry=(jnp.zeros((), jnp.float32), jnp.zeros((), jnp.float32),
                       jnp.full((), _NB - 1, jnp.int32),
                       jnp.zeros((), jnp.bool_),
                       jnp.ones((), jnp.float32), jnp.zeros((), jnp.float32)))
            def _scan(b, carry):
                cc, ss, bstar, found, ccs, sss = carry
                cc = cc + jnp.sum(hcnt[pl.ds(b * _L, _L)])
                ss = ss + jnp.sum(hsum[pl.ds(b * _L, _L)])
                t_edge = hi - (b + 1).astype(jnp.float32) * bw
                f = (s_top + ss) - (c_top + cc) * t_edge - 1.0
                hit = (f >= 0.0) & jnp.logical_not(found)
                bstar = jnp.where(hit, b, bstar)
                ccs = jnp.where(hit, c_top + cc, ccs)
                sss = jnp.where(hit, s_top + ss, sss)
                return cc, ss, bstar, found | hit, ccs, sss

            cc, ss, bstar, found, cc_star, ss_star = _scan
            cc_star = jnp.where(found, cc_star, c_top + cc)
            ss_star = jnp.where(found, ss_star, s_top + ss)
            bsf = bstar.astype(jnp.float32)
            lo = hi - (bsf + 3.0) * bw
            hi = hi - (bsf - 1.0) * bw

        tau = jnp.full((_L,), ss_star - 1.0) / jnp.full((_L,), cc_star)

        @plsc.parallel_loop(0, _NCH, unroll=8)
        def _out(i):
            v = buf[pl.ds(i * _L, _L)]
            buf[pl.ds(i * _L, _L)] = jnp.maximum(v - tau, 0.0)

        if copy_out is not None:
            copy_out.wait()
        copy_out = pltpu.async_copy(buf, out_hbm.at[row], sem_out)
    copy_out.wait()


def _make_sc_kernel():
    mesh = plsc.VectorSubcoreMesh(
        core_axis_name="c", subcore_axis_name="s",
        num_cores=2, num_subcores=16)
    return pl.kernel(
        _sc_body,
        out_type=jax.ShapeDtypeStruct((_R, _N), jnp.float32),
        mesh=mesh,
        scratch_types=[
            pltpu.VMEM(((_SC_ROWS // _NW) * _N,), jnp.float32),
            pltpu.VMEM((_NB * _L,), jnp.float32),
            pltpu.VMEM((_NB * _L,), jnp.float32),
            pltpu.SemaphoreType.DMA,
            pltpu.SemaphoreType.DMA,
        ],
        compiler_params=pltpu.CompilerParams(needs_layout_passes=False),
    )


def _tc_body(x_ref, o_ref):
    x = x_ref[...]
    m = jnp.max(x, axis=-1, keepdims=True)
    lo = m - 1.0
    hi = m

    def it(_, carry):
        lo, hi = carry
        mid = 0.5 * (lo + hi)
        s = jnp.sum(jnp.maximum(x - mid, 0.0), axis=-1, keepdims=True)
        pred = s > 1.0
        return jnp.where(pred, mid, lo), jnp.where(pred, hi, mid)

    lo, hi = jax.lax.fori_loop(0, _TC_ITERS, it, (lo, hi))
    sup = x > lo
    k = jnp.sum(sup.astype(jnp.float32), axis=-1, keepdims=True)
    s = jnp.sum(jnp.where(sup, x, 0.0), axis=-1, keepdims=True)
    tau = (s - 1.0) / k
    o_ref[...] = jnp.maximum(x - tau, 0.0)


def _tc_kernel(x_full):
    off = _SC_ROWS // _TC_BR
    return pl.pallas_call(
        _tc_body,
        grid=(_TC_ROWS // _TC_BR,),
        in_specs=[pl.BlockSpec((_TC_BR, _N), lambda i: (i + off, 0))],
        out_specs=pl.BlockSpec((_TC_BR, _N), lambda i: (i, 0)),
        out_shape=jax.ShapeDtypeStruct((_TC_ROWS, _N), jnp.float32),
    )(x_full)


@jax.jit
def kernel(input):
    sc_out = _make_sc_kernel()(input)
    tc_out = _tc_kernel(input)
    return lax.dynamic_update_slice(sc_out, tc_out, (_SC_ROWS, 0))
